# baseline (speedup 1.0000x reference)
import functools
import math

import jax
import jax.numpy as jnp
from jax.experimental import pallas as pl
from jax.experimental.pallas import tpu as pltpu

_HEADS = 8


def _dot_bt(a, b, bf):
    if bf:
        a, b = a.astype(jnp.bfloat16), b.astype(jnp.bfloat16)
    return jax.lax.dot_general(a, b, (((1,), (1,)), ((), ())),
                               preferred_element_type=jnp.float32)


def _dot(a, b, bf):
    if bf:
        a, b = a.astype(jnp.bfloat16), b.astype(jnp.bfloat16)
    return jax.lax.dot_general(a, b, (((1,), (0,)), ((), ())),
                               preferred_element_type=jnp.float32)


def _lin(x, p, bf=False):
    return _dot_bt(x, p['w'][...], bf) + p['b'][...]


def _ln(x, p):
    m = jnp.mean(x, axis=-1, keepdims=True)
    xc = x - m
    v = jnp.mean(xc * xc, axis=-1, keepdims=True)
    return xc * jax.lax.rsqrt(v + 1e-5) * p['g'][...] + p['b'][...]


def _ln_t(xt, g_col, b_col):
    m = jnp.mean(xt, axis=0, keepdims=True)
    xc = xt - m
    v = jnp.mean(xc * xc, axis=0, keepdims=True)
    return xc * jax.lax.rsqrt(v + 1e-5) * g_col + b_col


def _masks(C):
    hd = C // _HEADS
    lane = jax.lax.broadcasted_iota(jnp.int32, (1, C), 1)
    return [((lane >= h * hd) & (lane < (h + 1) * hd)).astype(jnp.float32)
            for h in range(_HEADS)]


def _smasks(C):
    hd = C // _HEADS
    sub = jax.lax.broadcasted_iota(jnp.int32, (C, 1), 0)
    return [((sub >= h * hd) & (sub < (h + 1) * hd)).astype(jnp.float32)
            for h in range(_HEADS)]


def _attn_self(p, q_in, k_in, v_in):
    q = _lin(q_in, p['q'])
    k = _lin(k_in, p['k'])
    v = _lin(v_in, p['v'])
    nq, C = q.shape
    hd = C // _HEADS
    scale = 1.0 / math.sqrt(hd)
    masks = _masks(C)
    qs = jnp.concatenate([q * (m * scale) for m in masks], axis=0)
    logits = _dot_bt(qs, k, False)
    e = jnp.exp(logits)
    a = e * (1.0 / jnp.sum(e, axis=-1, keepdims=True))
    oc = _dot(a, v, False)
    out = jnp.zeros((nq, C), jnp.float32)
    for h in range(_HEADS):
        out = out + oc[h * nq:(h + 1) * nq] * masks[h]
    return _lin(out, p['o'])


def _attn_t2i(p, q_in, kkt16, kt16):
    q = _lin(q_in, p['q'])
    nq, C = q.shape
    hd = C // _HEADS
    scale = 1.0 / math.sqrt(hd)
    masks = _masks(C)
    qs = jnp.concatenate([q * (m * scale) for m in masks], axis=0)
    kt = _dot(p['k']['w'][...], kkt16, True)
    vt = _dot(p['v']['w'][...], kt16, True)
    logits = _dot(qs, kt, True)
    e = jnp.exp(logits)
    a = e * (1.0 / jnp.sum(e, axis=-1, keepdims=True))
    oc = _dot_bt(a, vt, True)
    out = jnp.zeros((nq, C), jnp.float32)
    for h in range(_HEADS):
        out = out + oc[h * nq:(h + 1) * nq] * masks[h]
    return _lin(out, p['o'])


def _attn_i2t(p, kkt16, k_in, v_in, qb_col, ob_col):
    k = _lin(k_in, p['k'])
    v = jax.lax.dot_general(v_in, p['v']['w'][...],
                            (((1,), (1,)), ((), ())),
                            preferred_element_type=jnp.float32)
    nk, C = k.shape
    hd = C // _HEADS
    scale = 1.0 / math.sqrt(hd)
    masks = _masks(C)
    ks = jnp.concatenate([k * (m * scale) for m in masks], axis=0)
    qt = _dot(p['q']['w'][...], kkt16, True) + qb_col
    lt = _dot(ks, qt, True)
    e = jnp.exp(lt)
    ri = jax.lax.broadcasted_iota(jnp.int32, (_HEADS * nk, _HEADS * nk), 0)
    ci = jax.lax.broadcasted_iota(jnp.int32, (_HEADS * nk, _HEADS * nk), 1)
    seg = ((ri // nk) == (ci // nk)).astype(jnp.float32)
    d = _dot(seg, e, False)
    at = e * (1.0 / d)
    eye = (jax.lax.broadcasted_iota(jnp.int32, (nk, nk), 0)
           == jax.lax.broadcasted_iota(jnp.int32, (nk, nk), 1)
           ).astype(jnp.float32)
    vt = jax.lax.dot_general(v, eye, (((0,), (0,)), ((), ())),
                             preferred_element_type=jnp.float32)
    smasks = _smasks(C)
    vst = jnp.concatenate([vt * sm for sm in smasks], axis=1)
    out_t = _dot(vst, at, True)
    return _dot(p['o']['w'][...], out_t, True) + ob_col


def _body(treedef, n_param, xtreedef, n_extra, *refs):
    keys_ref, kpe_ref, point_ref = refs[:3]
    param_refs = refs[3:3 + n_param]
    extra_refs = refs[3 + n_param:3 + n_param + n_extra]
    q_out_ref, k_out_ref = refs[3 + n_param + n_extra:]
    p = jax.tree_util.tree_unflatten(treedef, list(param_refs))
    extra = jax.tree_util.tree_unflatten(xtreedef, list(extra_refs))

    keys_t = keys_ref[0]
    kpet16 = kpe_ref[0]
    point = point_ref[0]
    queries = point
    for i, bp in enumerate(p['blocks']):
        ex = extra['blocks'][i]
        if i == 0:
            queries = _attn_self(bp['self_attn'], queries, queries, queries)
        else:
            qq = queries + point
            queries = queries + _attn_self(bp['self_attn'], qq, qq, queries)
        queries = _ln(queries, bp['norm1'])
        qq = queries + point
        kt16 = keys_t.astype(jnp.bfloat16)
        kkt16 = kt16 + kpet16
        queries = queries + _attn_t2i(bp['cross_t2i'], qq, kkt16, kt16)
        queries = _ln(queries, bp['norm2'])
        h1 = jnp.maximum(_lin(queries, bp['mlp']['lin1']), 0.0)
        queries = queries + _lin(h1, bp['mlp']['lin2'])
        queries = _ln(queries, bp['norm3'])
        qq = queries + point
        keys_t = keys_t + _attn_i2t(bp['cross_i2t'], kkt16, qq, queries,
                                    ex['qb'][...], ex['ob'][...])
        keys_t = _ln_t(keys_t, ex['g4'][...], ex['b4'][...])
    qq = queries + point
    kt16 = keys_t.astype(jnp.bfloat16)
    kkt16 = kt16 + kpet16
    queries = queries + _attn_t2i(p['final_attn'], qq, kkt16, kt16)
    queries = _ln(queries, p['norm_final'])
    q_out_ref[0] = queries
    k_out_ref[0] = keys_t


@jax.jit
def kernel(image_embedding, image_pe, point_embedding, params):
    bs, c, h, w = image_embedding.shape
    n = h * w
    npt = point_embedding.shape[1]
    keys0 = image_embedding.reshape(bs, c, n)
    kpe0 = image_pe.reshape(bs, c, n).astype(jnp.bfloat16)

    params = jax.tree_util.tree_map(lambda x: x, params)
    extra = {'blocks': []}
    new_blocks = []
    for bp in params['blocks']:
        bp = dict(bp)
        for key in ('cross_t2i', 'cross_i2t'):
            at = dict(bp[key])
            o = dict(at['o'])
            o['b'] = o['b'] + at['v']['b'] @ o['w'].T
            at['o'] = o
            bp[key] = at
        extra['blocks'].append({
            'qb': bp['cross_i2t']['q']['b'].reshape(-1, 1),
            'ob': bp['cross_i2t']['o']['b'].reshape(-1, 1),
            'g4': bp['norm4']['g'].reshape(-1, 1),
            'b4': bp['norm4']['b'].reshape(-1, 1),
        })
        new_blocks.append(bp)
    params = dict(params)
    params['blocks'] = new_blocks
    fa = dict(params['final_attn'])
    fo = dict(fa['o'])
    fo['b'] = fo['b'] + fa['v']['b'] @ fo['w'].T
    fa['o'] = fo
    params['final_attn'] = fa

    flat, treedef = jax.tree_util.tree_flatten(params)
    flat = [f.reshape(1, -1) if f.ndim == 1 else f for f in flat]
    xflat, xtreedef = jax.tree_util.tree_flatten(extra)

    data_specs = [
        pl.BlockSpec((1, c, n), lambda b: (b, 0, 0)),
        pl.BlockSpec((1, c, n), lambda b: (b, 0, 0)),
        pl.BlockSpec((1, npt, c), lambda b: (b, 0, 0)),
    ]
    w_specs = [
        pl.BlockSpec(f.shape, lambda b, nd=f.ndim: (0,) * nd)
        for f in flat + xflat
    ]
    out_specs = [
        pl.BlockSpec((1, npt, c), lambda b: (b, 0, 0)),
        pl.BlockSpec((1, c, n), lambda b: (b, 0, 0)),
    ]
    out_shape = [
        jax.ShapeDtypeStruct((bs, npt, c), jnp.float32),
        jax.ShapeDtypeStruct((bs, c, n), jnp.float32),
    ]
    body = functools.partial(_body, treedef, len(flat), xtreedef, len(xflat))
    qs, ks_t = pl.pallas_call(
        body,
        grid=(bs,),
        in_specs=data_specs + w_specs,
        out_specs=out_specs,
        out_shape=out_shape,
        compiler_params=pltpu.CompilerParams(
            dimension_semantics=("arbitrary",),
        ),
    )(keys0, kpe0, point_embedding, *flat, *xflat)
    return qs, ks_t.transpose(0, 2, 1)
